# baseline (speedup 1.0000x reference)
import functools

import jax
import jax.numpy as jnp
from jax import lax
from jax.experimental import pallas as pl
from jax.experimental.pallas import tpu as pltpu
from jax.experimental.pallas import tpu_sc as plsc

B = 16384
D = 100
BLK = 2 * D
CHUNK = 32
SUB = 8
LANES = 16


def _sc_score_call():
    info = plsc.get_sparse_core_info()
    nc, ns = info.num_cores, info.num_subcores
    nw = nc * ns
    b_per_w = B // nw
    n_chunks = b_per_w // CHUNK
    mesh = plsc.VectorSubcoreMesh(core_axis_name="c", subcore_axis_name="s")

    @functools.partial(
        pl.kernel,
        out_type=(
            jax.ShapeDtypeStruct((B,), jnp.float32),
            jax.ShapeDtypeStruct((B,), jnp.float32),
        ),
        mesh=mesh,
        compiler_params=pltpu.CompilerParams(use_tc_tiling_on_sc=False,
                                             needs_layout_passes=False),
        scratch_types=[
            pltpu.VMEM((b_per_w,), jnp.int32),
            pltpu.VMEM((b_per_w,), jnp.int32),
            pltpu.VMEM((b_per_w,), jnp.int32),
            pltpu.VMEM((b_per_w,), jnp.int32),
            pltpu.VMEM((b_per_w,), jnp.int32),
            pltpu.VMEM((2, CHUNK, SUB, D), jnp.float32),
            pltpu.VMEM((2, CHUNK, BLK), jnp.float32),
            pltpu.VMEM((2, CHUNK, BLK), jnp.float32),
            pltpu.VMEM((CHUNK,), jnp.float32),
            pltpu.VMEM((CHUNK,), jnp.float32),
            pltpu.SemaphoreType.DMA,
            pltpu.SemaphoreType.DMA,
        ],
    )
    def sc_call(ui_hbm, pb_hbm, nb_hbm, po_hbm, no_hbm, ut_hbm, it_hbm,
                pos_hbm, neg_hbm,
                idx_u, idx_p, idx_n, off_p, off_n,
                tiles, p_rows, n_rows, pos_c, neg_c, s0, s1):
        wid = lax.axis_index("s") * nc + lax.axis_index("c")
        base_w = wid * b_per_w
        lane = lax.iota(jnp.int32, LANES)
        zeros = jnp.zeros((LANES,), jnp.float32)
        sems = (s0, s1)

        pltpu.sync_copy(ui_hbm.at[pl.ds(base_w, b_per_w)], idx_u)
        pltpu.sync_copy(pb_hbm.at[pl.ds(base_w, b_per_w)], idx_p)
        pltpu.sync_copy(nb_hbm.at[pl.ds(base_w, b_per_w)], idx_n)
        pltpu.sync_copy(po_hbm.at[pl.ds(base_w, b_per_w)], off_p)
        pltpu.sync_copy(no_hbm.at[pl.ds(base_w, b_per_w)], off_n)

        def issue(c, buf):
            vgs = [idx_u[pl.ds(c * CHUNK + k * LANES, LANES)]
                   for k in range(CHUNK // LANES)]
            for j in range(CHUNK):
                t = vgs[j // LANES][j % LANES] >> 3
                pltpu.async_copy(ut_hbm.at[t], tiles.at[buf, j], sems[buf])
            pltpu.async_copy(
                it_hbm.at[idx_p.at[pl.ds(c * CHUNK, CHUNK)]],
                p_rows.at[buf], sems[buf])
            pltpu.async_copy(
                it_hbm.at[idx_n.at[pl.ds(c * CHUNK, CHUNK)]],
                n_rows.at[buf], sems[buf])

        def drain(buf):
            for j in range(CHUNK):
                pltpu.make_async_copy(ut_hbm.at[0], tiles.at[buf, j],
                                      sems[buf]).wait()
            pltpu.make_async_copy(it_hbm.at[pl.ds(0, CHUNK)],
                                  p_rows.at[buf], sems[buf]).wait()
            pltpu.make_async_copy(it_hbm.at[pl.ds(0, CHUNK)],
                                  n_rows.at[buf], sems[buf]).wait()

        def compute(c, buf):
            base = base_w + c * CHUNK
            for g in range(CHUNK // LANES):
                rows = g * LANES + lane
                off = c * CHUNK + g * LANES
                subv = idx_u[pl.ds(off, LANES)] & 7
                ov_p = off_p[pl.ds(off, LANES)]
                ov_n = off_n[pl.ds(off, LANES)]

                def d_step(d, carry):
                    acc_p, acc_n, cp_, cn_ = carry
                    u = plsc.load_gather(
                        tiles.at[buf],
                        [rows, subv, jnp.full((LANES,), d, jnp.int32)])
                    p = plsc.load_gather(p_rows.at[buf], [rows, cp_])
                    n = plsc.load_gather(n_rows.at[buf], [rows, cn_])
                    return (acc_p + u * p, acc_n + u * n, cp_ + 1, cn_ + 1)

                acc_p, acc_n, _, _ = lax.fori_loop(
                    0, D, d_step, (zeros, zeros, ov_p, ov_n), unroll=4)
                pos_c[pl.ds(g * LANES, LANES)] = acc_p
                neg_c[pl.ds(g * LANES, LANES)] = acc_n
            pltpu.sync_copy(pos_c, pos_hbm.at[pl.ds(base, CHUNK)])
            pltpu.sync_copy(neg_c, neg_hbm.at[pl.ds(base, CHUNK)])

        issue(0, 0)
        issue(1, 1)

        def pair_body(p, _):
            for buf in range(2):
                c = 2 * p + buf
                drain(buf)
                compute(c, buf)

                @pl.when(p < n_chunks // 2 - 1)
                def _():
                    issue(c + 2, buf)

            return 0

        lax.fori_loop(0, n_chunks // 2, pair_body, 0)

    return sc_call


def kernel(user_inputs, pos_inputs, neg_inputs, user_table, item_table):
    ui = jnp.squeeze(user_inputs, axis=-1)
    pi = jnp.squeeze(pos_inputs, axis=-1)
    ni = jnp.squeeze(neg_inputs, axis=-1)
    ut3 = user_table.reshape(user_table.shape[0] // SUB, SUB, D)
    it2 = item_table.reshape(item_table.shape[0] // 2, BLK)
    pos, neg = _sc_score_call()(
        ui, pi >> 1, ni >> 1, (pi & 1) * D, (ni & 1) * D, ut3, it2)
    return (pos[:, None], neg[:, None])


# speedup vs baseline: 3.3756x; 3.3756x over previous
import functools

import jax
import jax.numpy as jnp
from jax import lax
from jax.experimental import pallas as pl
from jax.experimental.pallas import tpu as pltpu
from jax.experimental.pallas import tpu_sc as plsc

B = 16384
D = 100
CHUNK = 16
SUB = 8
LANES = 16


def _sc_score_call():
    info = plsc.get_sparse_core_info()
    nc, ns = info.num_cores, info.num_subcores
    nw = nc * ns
    b_per_w = B // nw
    n_chunks = b_per_w // CHUNK
    mesh = plsc.VectorSubcoreMesh(core_axis_name="c", subcore_axis_name="s")

    @functools.partial(
        pl.kernel,
        out_type=(
            jax.ShapeDtypeStruct((B,), jnp.float32),
            jax.ShapeDtypeStruct((B,), jnp.float32),
        ),
        mesh=mesh,
        compiler_params=pltpu.CompilerParams(use_tc_tiling_on_sc=True,
                                             needs_layout_passes=False),
        scratch_types=[
            pltpu.VMEM((b_per_w,), jnp.int32),
            pltpu.VMEM((b_per_w,), jnp.int32),
            pltpu.VMEM((b_per_w,), jnp.int32),
            pltpu.VMEM((2, CHUNK, SUB, D), jnp.float32),
            pltpu.VMEM((2, CHUNK, SUB, D), jnp.float32),
            pltpu.VMEM((2, CHUNK, SUB, D), jnp.float32),
            pltpu.VMEM((CHUNK,), jnp.float32),
            pltpu.VMEM((CHUNK,), jnp.float32),
            pltpu.SemaphoreType.DMA,
            pltpu.SemaphoreType.DMA,
        ],
    )
    def sc_call(ui_hbm, pi_hbm, ni_hbm, ut_hbm, it_hbm,
                pos_hbm, neg_hbm,
                idx_u, idx_p, idx_n,
                u_tiles, p_tiles, n_tiles, pos_c, neg_c, s0, s1):
        wid = lax.axis_index("s") * nc + lax.axis_index("c")
        base_w = wid * b_per_w
        lane = lax.iota(jnp.int32, LANES)
        zeros = jnp.zeros((LANES,), jnp.float32)
        sems = (s0, s1)

        pltpu.sync_copy(ui_hbm.at[pl.ds(base_w, b_per_w)], idx_u)
        pltpu.sync_copy(pi_hbm.at[pl.ds(base_w, b_per_w)], idx_p)
        pltpu.sync_copy(ni_hbm.at[pl.ds(base_w, b_per_w)], idx_n)

        def issue(c, buf):
            vgu = idx_u[pl.ds(c * CHUNK, CHUNK)]
            vgp = idx_p[pl.ds(c * CHUNK, CHUNK)]
            vgn = idx_n[pl.ds(c * CHUNK, CHUNK)]
            for j in range(CHUNK):
                pltpu.async_copy(ut_hbm.at[vgu[j] >> 3],
                                 u_tiles.at[buf, j], sems[buf])
                pltpu.async_copy(it_hbm.at[vgp[j] >> 3],
                                 p_tiles.at[buf, j], sems[buf])
                pltpu.async_copy(it_hbm.at[vgn[j] >> 3],
                                 n_tiles.at[buf, j], sems[buf])

        def drain(buf):
            for j in range(CHUNK):
                pltpu.make_async_copy(ut_hbm.at[0], u_tiles.at[buf, j],
                                      sems[buf]).wait()
                pltpu.make_async_copy(it_hbm.at[0], p_tiles.at[buf, j],
                                      sems[buf]).wait()
                pltpu.make_async_copy(it_hbm.at[0], n_tiles.at[buf, j],
                                      sems[buf]).wait()

        def compute(c, buf):
            base = base_w + c * CHUNK
            rows = lane
            sub_u = idx_u[pl.ds(c * CHUNK, CHUNK)] & 7
            sub_p = idx_p[pl.ds(c * CHUNK, CHUNK)] & 7
            sub_n = idx_n[pl.ds(c * CHUNK, CHUNK)] & 7

            def d_step(d, carry):
                acc_p, acc_n = carry
                dv = jnp.full((LANES,), d, jnp.int32)
                u = plsc.load_gather(u_tiles.at[buf], [rows, sub_u, dv])
                p = plsc.load_gather(p_tiles.at[buf], [rows, sub_p, dv])
                n = plsc.load_gather(n_tiles.at[buf], [rows, sub_n, dv])
                return (acc_p + u * p, acc_n + u * n)

            acc_p, acc_n = lax.fori_loop(
                0, D, d_step, (zeros, zeros), unroll=4)
            pos_c[...] = acc_p
            neg_c[...] = acc_n
            pltpu.sync_copy(pos_c, pos_hbm.at[pl.ds(base, CHUNK)])
            pltpu.sync_copy(neg_c, neg_hbm.at[pl.ds(base, CHUNK)])

        issue(0, 0)
        issue(1, 1)

        def pair_body(p, _):
            for buf in range(2):
                c = 2 * p + buf
                drain(buf)
                compute(c, buf)

                @pl.when(p < n_chunks // 2 - 1)
                def _():
                    issue(c + 2, buf)

            return 0

        lax.fori_loop(0, n_chunks // 2, pair_body, 0)

    return sc_call


def kernel(user_inputs, pos_inputs, neg_inputs, user_table, item_table):
    ui = jnp.squeeze(user_inputs, axis=-1)
    pi = jnp.squeeze(pos_inputs, axis=-1)
    ni = jnp.squeeze(neg_inputs, axis=-1)
    ut3 = user_table.reshape(user_table.shape[0] // SUB, SUB, D)
    it3 = item_table.reshape(item_table.shape[0] // SUB, SUB, D)
    pos, neg = _sc_score_call()(ui, pi, ni, ut3, it3)
    return (pos[:, None], neg[:, None])


# speedup vs baseline: 3.4009x; 1.0075x over previous
import functools

import jax
import jax.numpy as jnp
from jax import lax
from jax.experimental import pallas as pl
from jax.experimental.pallas import tpu as pltpu
from jax.experimental.pallas import tpu_sc as plsc

B = 16384
D = 100
BLK = 2 * D
CHUNK = 128
GCHUNK = 16
SUB = 8
LANES = 16


def _sc_user_gather_call():
    info = plsc.get_sparse_core_info()
    nc, ns = info.num_cores, info.num_subcores
    nw = nc * ns
    b_per_w = B // nw
    n_chunks = b_per_w // GCHUNK
    mesh = plsc.VectorSubcoreMesh(core_axis_name="c", subcore_axis_name="s")

    @functools.partial(
        pl.kernel,
        out_type=jax.ShapeDtypeStruct((B * D,), jnp.float32),
        mesh=mesh,
        compiler_params=pltpu.CompilerParams(use_tc_tiling_on_sc=True,
                                             needs_layout_passes=False),
        scratch_types=[
            pltpu.VMEM((b_per_w,), jnp.int32),
            pltpu.VMEM((2, GCHUNK, SUB, D), jnp.float32),
            pltpu.VMEM((b_per_w * D,), jnp.float32),
            pltpu.SemaphoreType.DMA,
            pltpu.SemaphoreType.DMA,
        ],
    )
    def gather_call(ui_hbm, ut_hbm, uf_hbm, idx_u, tiles, u_loc, s0, s1):
        wid = lax.axis_index("s") * nc + lax.axis_index("c")
        base_w = wid * b_per_w
        sems = (s0, s1)

        pltpu.sync_copy(ui_hbm.at[pl.ds(base_w, b_per_w)], idx_u)

        def issue(c, buf):
            vg = idx_u[pl.ds(c * GCHUNK, GCHUNK)]
            for j in range(GCHUNK):
                pltpu.async_copy(ut_hbm.at[vg[j] >> 3],
                                 tiles.at[buf, j], sems[buf])

        def drain(buf):
            for j in range(GCHUNK):
                pltpu.make_async_copy(ut_hbm.at[0], tiles.at[buf, j],
                                      sems[buf]).wait()

        def repack(c, buf):
            vg = idx_u[pl.ds(c * GCHUNK, GCHUNK)]
            for j in range(GCHUNK):
                s = vg[j] & 7
                dst = (c * GCHUNK + j) * D
                for k in range(D // LANES):
                    u_loc[pl.ds(dst + k * LANES, LANES)] = (
                        tiles[buf, j, s, pl.ds(k * LANES, LANES)])
                u_loc[pl.ds(dst + D - LANES, LANES)] = (
                    tiles[buf, j, s, pl.ds(D - LANES, LANES)])

        issue(0, 0)
        issue(1, 1)

        def pair_body(p, _):
            for buf in range(2):
                c = 2 * p + buf
                drain(buf)
                repack(c, buf)

                @pl.when(p < n_chunks // 2 - 1)
                def _():
                    issue(c + 2, buf)

            return 0

        lax.fori_loop(0, n_chunks // 2, pair_body, 0)
        pltpu.sync_copy(u_loc, uf_hbm.at[pl.ds(base_w * D, b_per_w * D)])

    return gather_call


def _sc_score_call():
    info = plsc.get_sparse_core_info()
    nc, ns = info.num_cores, info.num_subcores
    nw = nc * ns
    b_per_w = B // nw
    n_chunks = b_per_w // CHUNK
    mesh = plsc.VectorSubcoreMesh(core_axis_name="c", subcore_axis_name="s")

    @functools.partial(
        pl.kernel,
        out_type=(
            jax.ShapeDtypeStruct((B,), jnp.float32),
            jax.ShapeDtypeStruct((B,), jnp.float32),
        ),
        mesh=mesh,
        compiler_params=pltpu.CompilerParams(use_tc_tiling_on_sc=False,
                                             needs_layout_passes=False),
        scratch_types=[
            pltpu.VMEM((b_per_w * D,), jnp.float32),
            pltpu.VMEM((CHUNK,), jnp.int32),
            pltpu.VMEM((CHUNK,), jnp.int32),
            pltpu.VMEM((CHUNK,), jnp.int32),
            pltpu.VMEM((CHUNK,), jnp.int32),
            pltpu.VMEM((CHUNK, BLK), jnp.float32),
            pltpu.VMEM((CHUNK, BLK), jnp.float32),
            pltpu.VMEM((CHUNK,), jnp.float32),
            pltpu.VMEM((CHUNK,), jnp.float32),
            pltpu.SemaphoreType.DMA,
            pltpu.SemaphoreType.DMA,
        ],
    )
    def sc_call(pb_hbm, nb_hbm, po_hbm, no_hbm, it_hbm, uf_hbm,
                pos_hbm, neg_hbm,
                u_loc, idx_p, idx_n, off_p, off_n, p_rows, n_rows,
                pos_c, neg_c, sem_u, sem):
        wid = lax.axis_index("s") * nc + lax.axis_index("c")
        lane = lax.iota(jnp.int32, LANES)
        zeros = jnp.zeros((LANES,), jnp.float32)

        cu = pltpu.async_copy(
            uf_hbm.at[pl.ds(wid * b_per_w * D, b_per_w * D)], u_loc, sem_u)

        for c in range(n_chunks):
            base = wid * b_per_w + c * CHUNK
            pltpu.sync_copy(pb_hbm.at[pl.ds(base, CHUNK)], idx_p)
            pltpu.sync_copy(nb_hbm.at[pl.ds(base, CHUNK)], idx_n)
            pltpu.sync_copy(po_hbm.at[pl.ds(base, CHUNK)], off_p)
            pltpu.sync_copy(no_hbm.at[pl.ds(base, CHUNK)], off_n)
            cp = pltpu.async_copy(it_hbm.at[idx_p], p_rows, sem)
            cn = pltpu.async_copy(it_hbm.at[idx_n], n_rows, sem)
            cp.wait()
            cn.wait()
            if c == 0:
                cu.wait()

            def group_body(g, _):
                rows = g * LANES + lane
                ov_p = off_p[pl.ds(g * LANES, LANES)]
                ov_n = off_n[pl.ds(g * LANES, LANES)]
                u_idx0 = (c * CHUNK + rows) * D

                def d_step(d, carry):
                    acc_p, acc_n, ui_, cp_, cn_ = carry
                    u = plsc.load_gather(u_loc, [ui_])
                    p = plsc.load_gather(p_rows, [rows, cp_])
                    n = plsc.load_gather(n_rows, [rows, cn_])
                    return (acc_p + u * p, acc_n + u * n,
                            ui_ + 1, cp_ + 1, cn_ + 1)

                acc_p, acc_n, _, _, _ = lax.fori_loop(
                    0, D, d_step, (zeros, zeros, u_idx0, ov_p, ov_n),
                    unroll=4)
                pos_c[pl.ds(g * LANES, LANES)] = acc_p
                neg_c[pl.ds(g * LANES, LANES)] = acc_n
                return 0

            lax.fori_loop(0, CHUNK // LANES, group_body, 0)
            pltpu.sync_copy(pos_c, pos_hbm.at[pl.ds(base, CHUNK)])
            pltpu.sync_copy(neg_c, neg_hbm.at[pl.ds(base, CHUNK)])

    return sc_call


def kernel(user_inputs, pos_inputs, neg_inputs, user_table, item_table):
    ui = jnp.squeeze(user_inputs, axis=-1)
    pi = jnp.squeeze(pos_inputs, axis=-1)
    ni = jnp.squeeze(neg_inputs, axis=-1)
    ut3 = user_table.reshape(user_table.shape[0] // SUB, SUB, D)
    u_flat = _sc_user_gather_call()(ui, ut3)
    it2 = item_table.reshape(item_table.shape[0] // 2, BLK)
    pos, neg = _sc_score_call()(
        pi >> 1, ni >> 1, (pi & 1) * D, (ni & 1) * D, it2, u_flat)
    return (pos[:, None], neg[:, None])


# speedup vs baseline: 3.4075x; 1.0019x over previous
import functools

import jax
import jax.numpy as jnp
from jax import lax
from jax.experimental import pallas as pl
from jax.experimental.pallas import tpu as pltpu
from jax.experimental.pallas import tpu_sc as plsc

B = 16384
D = 100
BLK = 2 * D
CHUNK = 128
GCHUNK = 32
SUB = 8
LANES = 16


def _sc_user_gather_call():
    info = plsc.get_sparse_core_info()
    nc, ns = info.num_cores, info.num_subcores
    nw = nc * ns
    b_per_w = B // nw
    n_chunks = b_per_w // GCHUNK
    mesh = plsc.VectorSubcoreMesh(core_axis_name="c", subcore_axis_name="s")

    @functools.partial(
        pl.kernel,
        out_type=jax.ShapeDtypeStruct((B * D,), jnp.float32),
        mesh=mesh,
        compiler_params=pltpu.CompilerParams(use_tc_tiling_on_sc=True,
                                             needs_layout_passes=False),
        scratch_types=[
            pltpu.VMEM((b_per_w,), jnp.int32),
            pltpu.VMEM((2, GCHUNK, SUB, D), jnp.float32),
            pltpu.VMEM((b_per_w * D,), jnp.float32),
            pltpu.SemaphoreType.DMA,
            pltpu.SemaphoreType.DMA,
        ],
    )
    def gather_call(ui_hbm, ut_hbm, uf_hbm, idx_u, tiles, u_loc, s0, s1):
        wid = lax.axis_index("s") * nc + lax.axis_index("c")
        base_w = wid * b_per_w
        sems = (s0, s1)

        pltpu.sync_copy(ui_hbm.at[pl.ds(base_w, b_per_w)], idx_u)

        def issue(c, buf):
            vgs = [idx_u[pl.ds(c * GCHUNK + k * LANES, LANES)]
                   for k in range(GCHUNK // LANES)]
            for j in range(GCHUNK):
                pltpu.async_copy(ut_hbm.at[vgs[j // LANES][j % LANES] >> 3],
                                 tiles.at[buf, j], sems[buf])

        def drain(buf):
            for j in range(GCHUNK):
                pltpu.make_async_copy(ut_hbm.at[0], tiles.at[buf, j],
                                      sems[buf]).wait()

        def repack(c, buf):
            vgs = [idx_u[pl.ds(c * GCHUNK + k * LANES, LANES)]
                   for k in range(GCHUNK // LANES)]
            for j in range(GCHUNK):
                s = vgs[j // LANES][j % LANES] & 7
                dst = (c * GCHUNK + j) * D
                for k in range(D // LANES):
                    u_loc[pl.ds(dst + k * LANES, LANES)] = (
                        tiles[buf, j, s, pl.ds(k * LANES, LANES)])
                u_loc[pl.ds(dst + D - LANES, LANES)] = (
                    tiles[buf, j, s, pl.ds(D - LANES, LANES)])

        issue(0, 0)
        issue(1, 1)

        def pair_body(p, _):
            for buf in range(2):
                c = 2 * p + buf
                drain(buf)
                repack(c, buf)

                @pl.when(p < n_chunks // 2 - 1)
                def _():
                    issue(c + 2, buf)

            return 0

        lax.fori_loop(0, n_chunks // 2, pair_body, 0)
        pltpu.sync_copy(u_loc, uf_hbm.at[pl.ds(base_w * D, b_per_w * D)])

    return gather_call


def _sc_score_call():
    info = plsc.get_sparse_core_info()
    nc, ns = info.num_cores, info.num_subcores
    nw = nc * ns
    b_per_w = B // nw
    n_chunks = b_per_w // CHUNK
    mesh = plsc.VectorSubcoreMesh(core_axis_name="c", subcore_axis_name="s")

    @functools.partial(
        pl.kernel,
        out_type=(
            jax.ShapeDtypeStruct((B,), jnp.float32),
            jax.ShapeDtypeStruct((B,), jnp.float32),
        ),
        mesh=mesh,
        compiler_params=pltpu.CompilerParams(use_tc_tiling_on_sc=False,
                                             needs_layout_passes=False),
        scratch_types=[
            pltpu.VMEM((b_per_w * D,), jnp.float32),
            pltpu.VMEM((CHUNK,), jnp.int32),
            pltpu.VMEM((CHUNK,), jnp.int32),
            pltpu.VMEM((CHUNK,), jnp.int32),
            pltpu.VMEM((CHUNK,), jnp.int32),
            pltpu.VMEM((CHUNK, BLK), jnp.float32),
            pltpu.VMEM((CHUNK, BLK), jnp.float32),
            pltpu.VMEM((CHUNK,), jnp.float32),
            pltpu.VMEM((CHUNK,), jnp.float32),
            pltpu.SemaphoreType.DMA,
            pltpu.SemaphoreType.DMA,
        ],
    )
    def sc_call(pb_hbm, nb_hbm, po_hbm, no_hbm, it_hbm, uf_hbm,
                pos_hbm, neg_hbm,
                u_loc, idx_p, idx_n, off_p, off_n, p_rows, n_rows,
                pos_c, neg_c, sem_u, sem):
        wid = lax.axis_index("s") * nc + lax.axis_index("c")
        lane = lax.iota(jnp.int32, LANES)
        zeros = jnp.zeros((LANES,), jnp.float32)

        cu = pltpu.async_copy(
            uf_hbm.at[pl.ds(wid * b_per_w * D, b_per_w * D)], u_loc, sem_u)

        for c in range(n_chunks):
            base = wid * b_per_w + c * CHUNK
            pltpu.sync_copy(pb_hbm.at[pl.ds(base, CHUNK)], idx_p)
            pltpu.sync_copy(nb_hbm.at[pl.ds(base, CHUNK)], idx_n)
            pltpu.sync_copy(po_hbm.at[pl.ds(base, CHUNK)], off_p)
            pltpu.sync_copy(no_hbm.at[pl.ds(base, CHUNK)], off_n)
            cp = pltpu.async_copy(it_hbm.at[idx_p], p_rows, sem)
            cn = pltpu.async_copy(it_hbm.at[idx_n], n_rows, sem)
            cp.wait()
            cn.wait()
            if c == 0:
                cu.wait()

            def group_body(g, _):
                rows = g * LANES + lane
                ov_p = off_p[pl.ds(g * LANES, LANES)]
                ov_n = off_n[pl.ds(g * LANES, LANES)]
                u_idx0 = (c * CHUNK + rows) * D

                def d_step(d, carry):
                    acc_p, acc_n, ui_, cp_, cn_ = carry
                    u = plsc.load_gather(u_loc, [ui_])
                    p = plsc.load_gather(p_rows, [rows, cp_])
                    n = plsc.load_gather(n_rows, [rows, cn_])
                    return (acc_p + u * p, acc_n + u * n,
                            ui_ + 1, cp_ + 1, cn_ + 1)

                acc_p, acc_n, _, _, _ = lax.fori_loop(
                    0, D, d_step, (zeros, zeros, u_idx0, ov_p, ov_n),
                    unroll=4)
                pos_c[pl.ds(g * LANES, LANES)] = acc_p
                neg_c[pl.ds(g * LANES, LANES)] = acc_n
                return 0

            lax.fori_loop(0, CHUNK // LANES, group_body, 0)
            pltpu.sync_copy(pos_c, pos_hbm.at[pl.ds(base, CHUNK)])
            pltpu.sync_copy(neg_c, neg_hbm.at[pl.ds(base, CHUNK)])

    return sc_call


def kernel(user_inputs, pos_inputs, neg_inputs, user_table, item_table):
    ui = jnp.squeeze(user_inputs, axis=-1)
    pi = jnp.squeeze(pos_inputs, axis=-1)
    ni = jnp.squeeze(neg_inputs, axis=-1)
    ut3 = user_table.reshape(user_table.shape[0] // SUB, SUB, D)
    u_flat = _sc_user_gather_call()(ui, ut3)
    it2 = item_table.reshape(item_table.shape[0] // 2, BLK)
    pos, neg = _sc_score_call()(
        pi >> 1, ni >> 1, (pi & 1) * D, (ni & 1) * D, it2, u_flat)
    return (pos[:, None], neg[:, None])


# speedup vs baseline: 3.4156x; 1.0024x over previous
import functools

import jax
import jax.numpy as jnp
from jax import lax
from jax.experimental import pallas as pl
from jax.experimental.pallas import tpu as pltpu
from jax.experimental.pallas import tpu_sc as plsc

B = 16384
D = 100
BLK = 2 * D
CHUNK = 128
GCHUNK = 32
SUB = 8
LANES = 16


def _sc_user_gather_call():
    info = plsc.get_sparse_core_info()
    nc, ns = info.num_cores, info.num_subcores
    nw = nc * ns
    b_per_w = B // nw
    n_chunks = b_per_w // GCHUNK
    mesh = plsc.VectorSubcoreMesh(core_axis_name="c", subcore_axis_name="s")

    @functools.partial(
        pl.kernel,
        out_type=jax.ShapeDtypeStruct((B * D,), jnp.float32),
        mesh=mesh,
        compiler_params=pltpu.CompilerParams(use_tc_tiling_on_sc=True,
                                             needs_layout_passes=False),
        scratch_types=[
            pltpu.VMEM((b_per_w,), jnp.int32),
            pltpu.VMEM((2, GCHUNK, 1, D), jnp.float32),
            pltpu.VMEM((b_per_w * D,), jnp.float32),
            pltpu.SemaphoreType.DMA,
            pltpu.SemaphoreType.DMA,
        ],
    )
    def gather_call(ui_hbm, ut_hbm, uf_hbm, idx_u, tiles, u_loc, s0, s1):
        wid = lax.axis_index("s") * nc + lax.axis_index("c")
        base_w = wid * b_per_w
        sems = (s0, s1)

        pltpu.sync_copy(ui_hbm.at[pl.ds(base_w, b_per_w)], idx_u)

        def issue(c, buf):
            vgs = [idx_u[pl.ds(c * GCHUNK + k * LANES, LANES)]
                   for k in range(GCHUNK // LANES)]
            for j in range(GCHUNK):
                pltpu.async_copy(ut_hbm.at[vgs[j // LANES][j % LANES]],
                                 tiles.at[buf, j], sems[buf])

        def drain(buf):
            for j in range(GCHUNK):
                pltpu.make_async_copy(ut_hbm.at[0], tiles.at[buf, j],
                                      sems[buf]).wait()

        def repack(c, buf):
            vgs = [idx_u[pl.ds(c * GCHUNK + k * LANES, LANES)]
                   for k in range(GCHUNK // LANES)]
            for j in range(GCHUNK):
                dst = (c * GCHUNK + j) * D
                for k in range(D // LANES):
                    u_loc[pl.ds(dst + k * LANES, LANES)] = (
                        tiles[buf, j, 0, pl.ds(k * LANES, LANES)])
                u_loc[pl.ds(dst + D - LANES, LANES)] = (
                    tiles[buf, j, 0, pl.ds(D - LANES, LANES)])

        issue(0, 0)
        issue(1, 1)

        def pair_body(p, _):
            for buf in range(2):
                c = 2 * p + buf
                drain(buf)
                repack(c, buf)

                @pl.when(p < n_chunks // 2 - 1)
                def _():
                    issue(c + 2, buf)

            return 0

        lax.fori_loop(0, n_chunks // 2, pair_body, 0)
        pltpu.sync_copy(u_loc, uf_hbm.at[pl.ds(base_w * D, b_per_w * D)])

    return gather_call


def _sc_score_call():
    info = plsc.get_sparse_core_info()
    nc, ns = info.num_cores, info.num_subcores
    nw = nc * ns
    b_per_w = B // nw
    n_chunks = b_per_w // CHUNK
    mesh = plsc.VectorSubcoreMesh(core_axis_name="c", subcore_axis_name="s")

    @functools.partial(
        pl.kernel,
        out_type=(
            jax.ShapeDtypeStruct((B,), jnp.float32),
            jax.ShapeDtypeStruct((B,), jnp.float32),
        ),
        mesh=mesh,
        compiler_params=pltpu.CompilerParams(use_tc_tiling_on_sc=False,
                                             needs_layout_passes=False),
        scratch_types=[
            pltpu.VMEM((b_per_w * D,), jnp.float32),
            pltpu.VMEM((CHUNK,), jnp.int32),
            pltpu.VMEM((CHUNK,), jnp.int32),
            pltpu.VMEM((CHUNK,), jnp.int32),
            pltpu.VMEM((CHUNK,), jnp.int32),
            pltpu.VMEM((CHUNK, BLK), jnp.float32),
            pltpu.VMEM((CHUNK, BLK), jnp.float32),
            pltpu.VMEM((CHUNK,), jnp.float32),
            pltpu.VMEM((CHUNK,), jnp.float32),
            pltpu.SemaphoreType.DMA,
            pltpu.SemaphoreType.DMA,
        ],
    )
    def sc_call(pb_hbm, nb_hbm, po_hbm, no_hbm, it_hbm, uf_hbm,
                pos_hbm, neg_hbm,
                u_loc, idx_p, idx_n, off_p, off_n, p_rows, n_rows,
                pos_c, neg_c, sem_u, sem):
        wid = lax.axis_index("s") * nc + lax.axis_index("c")
        lane = lax.iota(jnp.int32, LANES)
        zeros = jnp.zeros((LANES,), jnp.float32)

        cu = pltpu.async_copy(
            uf_hbm.at[pl.ds(wid * b_per_w * D, b_per_w * D)], u_loc, sem_u)

        for c in range(n_chunks):
            base = wid * b_per_w + c * CHUNK
            pltpu.sync_copy(pb_hbm.at[pl.ds(base, CHUNK)], idx_p)
            pltpu.sync_copy(nb_hbm.at[pl.ds(base, CHUNK)], idx_n)
            pltpu.sync_copy(po_hbm.at[pl.ds(base, CHUNK)], off_p)
            pltpu.sync_copy(no_hbm.at[pl.ds(base, CHUNK)], off_n)
            cp = pltpu.async_copy(it_hbm.at[idx_p], p_rows, sem)
            cn = pltpu.async_copy(it_hbm.at[idx_n], n_rows, sem)
            cp.wait()
            cn.wait()
            if c == 0:
                cu.wait()

            def group_body(g, _):
                rows = g * LANES + lane
                ov_p = off_p[pl.ds(g * LANES, LANES)]
                ov_n = off_n[pl.ds(g * LANES, LANES)]
                u_idx0 = (c * CHUNK + rows) * D

                def d_step(d, carry):
                    acc_p, acc_n, ui_, cp_, cn_ = carry
                    u = plsc.load_gather(u_loc, [ui_])
                    p = plsc.load_gather(p_rows, [rows, cp_])
                    n = plsc.load_gather(n_rows, [rows, cn_])
                    return (acc_p + u * p, acc_n + u * n,
                            ui_ + 1, cp_ + 1, cn_ + 1)

                acc_p, acc_n, _, _, _ = lax.fori_loop(
                    0, D, d_step, (zeros, zeros, u_idx0, ov_p, ov_n),
                    unroll=4)
                pos_c[pl.ds(g * LANES, LANES)] = acc_p
                neg_c[pl.ds(g * LANES, LANES)] = acc_n
                return 0

            lax.fori_loop(0, CHUNK // LANES, group_body, 0)
            pltpu.sync_copy(pos_c, pos_hbm.at[pl.ds(base, CHUNK)])
            pltpu.sync_copy(neg_c, neg_hbm.at[pl.ds(base, CHUNK)])

    return sc_call


def kernel(user_inputs, pos_inputs, neg_inputs, user_table, item_table):
    ui = jnp.squeeze(user_inputs, axis=-1)
    pi = jnp.squeeze(pos_inputs, axis=-1)
    ni = jnp.squeeze(neg_inputs, axis=-1)
    ut3 = user_table.reshape(user_table.shape[0], 1, D)
    u_flat = _sc_user_gather_call()(ui, ut3)
    it2 = item_table.reshape(item_table.shape[0] // 2, BLK)
    pos, neg = _sc_score_call()(
        pi >> 1, ni >> 1, (pi & 1) * D, (ni & 1) * D, it2, u_flat)
    return (pos[:, None], neg[:, None])


# speedup vs baseline: 3.4488x; 1.0097x over previous
import functools

import jax
import jax.numpy as jnp
from jax import lax
from jax.experimental import pallas as pl
from jax.experimental.pallas import tpu as pltpu
from jax.experimental.pallas import tpu_sc as plsc

B = 16384
D = 100
BLK = 2 * D
CHUNK = 64
GCHUNK = 32
SUB = 8
LANES = 16


def _sc_user_gather_call():
    info = plsc.get_sparse_core_info()
    nc, ns = info.num_cores, info.num_subcores
    nw = nc * ns
    b_per_w = B // nw
    n_chunks = b_per_w // GCHUNK
    mesh = plsc.VectorSubcoreMesh(core_axis_name="c", subcore_axis_name="s")

    @functools.partial(
        pl.kernel,
        out_type=jax.ShapeDtypeStruct((B * D,), jnp.float32),
        mesh=mesh,
        compiler_params=pltpu.CompilerParams(use_tc_tiling_on_sc=True,
                                             needs_layout_passes=False),
        scratch_types=[
            pltpu.VMEM((b_per_w,), jnp.int32),
            pltpu.VMEM((2, GCHUNK, 1, D), jnp.float32),
            pltpu.VMEM((b_per_w * D,), jnp.float32),
            pltpu.SemaphoreType.DMA,
            pltpu.SemaphoreType.DMA,
        ],
    )
    def gather_call(ui_hbm, ut_hbm, uf_hbm, idx_u, tiles, u_loc, s0, s1):
        wid = lax.axis_index("s") * nc + lax.axis_index("c")
        base_w = wid * b_per_w
        sems = (s0, s1)

        pltpu.sync_copy(ui_hbm.at[pl.ds(base_w, b_per_w)], idx_u)

        def issue(c, buf):
            vgs = [idx_u[pl.ds(c * GCHUNK + k * LANES, LANES)]
                   for k in range(GCHUNK // LANES)]
            for j in range(GCHUNK):
                pltpu.async_copy(ut_hbm.at[vgs[j // LANES][j % LANES]],
                                 tiles.at[buf, j], sems[buf])

        def drain(buf):
            for j in range(GCHUNK):
                pltpu.make_async_copy(ut_hbm.at[0], tiles.at[buf, j],
                                      sems[buf]).wait()

        def repack(c, buf):
            vgs = [idx_u[pl.ds(c * GCHUNK + k * LANES, LANES)]
                   for k in range(GCHUNK // LANES)]
            for j in range(GCHUNK):
                dst = (c * GCHUNK + j) * D
                for k in range(D // LANES):
                    u_loc[pl.ds(dst + k * LANES, LANES)] = (
                        tiles[buf, j, 0, pl.ds(k * LANES, LANES)])
                u_loc[pl.ds(dst + D - LANES, LANES)] = (
                    tiles[buf, j, 0, pl.ds(D - LANES, LANES)])

        issue(0, 0)
        issue(1, 1)

        def pair_body(p, _):
            for buf in range(2):
                c = 2 * p + buf
                drain(buf)
                repack(c, buf)

                @pl.when(p < n_chunks // 2 - 1)
                def _():
                    issue(c + 2, buf)

            return 0

        lax.fori_loop(0, n_chunks // 2, pair_body, 0)
        pltpu.sync_copy(u_loc, uf_hbm.at[pl.ds(base_w * D, b_per_w * D)])

    return gather_call


def _sc_score_call():
    info = plsc.get_sparse_core_info()
    nc, ns = info.num_cores, info.num_subcores
    nw = nc * ns
    b_per_w = B // nw
    n_chunks = b_per_w // CHUNK
    mesh = plsc.VectorSubcoreMesh(core_axis_name="c", subcore_axis_name="s")

    @functools.partial(
        pl.kernel,
        out_type=(
            jax.ShapeDtypeStruct((B,), jnp.float32),
            jax.ShapeDtypeStruct((B,), jnp.float32),
        ),
        mesh=mesh,
        compiler_params=pltpu.CompilerParams(use_tc_tiling_on_sc=False,
                                             needs_layout_passes=False),
        scratch_types=[
            pltpu.VMEM((b_per_w * D,), jnp.float32),
            pltpu.VMEM((b_per_w,), jnp.int32),
            pltpu.VMEM((b_per_w,), jnp.int32),
            pltpu.VMEM((b_per_w,), jnp.int32),
            pltpu.VMEM((b_per_w,), jnp.int32),
            pltpu.VMEM((2, CHUNK, BLK), jnp.float32),
            pltpu.VMEM((2, CHUNK, BLK), jnp.float32),
            pltpu.VMEM((CHUNK,), jnp.float32),
            pltpu.VMEM((CHUNK,), jnp.float32),
            pltpu.SemaphoreType.DMA,
            pltpu.SemaphoreType.DMA,
            pltpu.SemaphoreType.DMA,
        ],
    )
    def sc_call(pb_hbm, nb_hbm, po_hbm, no_hbm, it_hbm, uf_hbm,
                pos_hbm, neg_hbm,
                u_loc, idx_p, idx_n, off_p, off_n, p_rows, n_rows,
                pos_c, neg_c, sem_u, s0, s1):
        wid = lax.axis_index("s") * nc + lax.axis_index("c")
        base_w = wid * b_per_w
        lane = lax.iota(jnp.int32, LANES)
        zeros = jnp.zeros((LANES,), jnp.float32)
        sems = (s0, s1)

        cu = pltpu.async_copy(
            uf_hbm.at[pl.ds(base_w * D, b_per_w * D)], u_loc, sem_u)
        pltpu.sync_copy(pb_hbm.at[pl.ds(base_w, b_per_w)], idx_p)
        pltpu.sync_copy(nb_hbm.at[pl.ds(base_w, b_per_w)], idx_n)
        pltpu.sync_copy(po_hbm.at[pl.ds(base_w, b_per_w)], off_p)
        pltpu.sync_copy(no_hbm.at[pl.ds(base_w, b_per_w)], off_n)

        def issue(c, buf):
            pltpu.async_copy(
                it_hbm.at[idx_p.at[pl.ds(c * CHUNK, CHUNK)]],
                p_rows.at[buf], sems[buf])
            pltpu.async_copy(
                it_hbm.at[idx_n.at[pl.ds(c * CHUNK, CHUNK)]],
                n_rows.at[buf], sems[buf])

        def drain(buf):
            pltpu.make_async_copy(it_hbm.at[pl.ds(0, CHUNK)],
                                  p_rows.at[buf], sems[buf]).wait()
            pltpu.make_async_copy(it_hbm.at[pl.ds(0, CHUNK)],
                                  n_rows.at[buf], sems[buf]).wait()

        def compute(c, buf):
            base = base_w + c * CHUNK
            for g in range(CHUNK // LANES):
                rows = g * LANES + lane
                off = c * CHUNK + g * LANES
                ov_p = off_p[pl.ds(off, LANES)]
                ov_n = off_n[pl.ds(off, LANES)]
                u_idx0 = (off + lane) * D

                def d_step(d, carry):
                    acc_p, acc_n, ui_, cp_, cn_ = carry
                    u = plsc.load_gather(u_loc, [ui_])
                    p = plsc.load_gather(p_rows.at[buf], [rows, cp_])
                    n = plsc.load_gather(n_rows.at[buf], [rows, cn_])
                    return (acc_p + u * p, acc_n + u * n,
                            ui_ + 1, cp_ + 1, cn_ + 1)

                acc_p, acc_n, _, _, _ = lax.fori_loop(
                    0, D, d_step, (zeros, zeros, u_idx0, ov_p, ov_n),
                    unroll=4)
                pos_c[pl.ds(g * LANES, LANES)] = acc_p
                neg_c[pl.ds(g * LANES, LANES)] = acc_n
            pltpu.sync_copy(pos_c, pos_hbm.at[pl.ds(base, CHUNK)])
            pltpu.sync_copy(neg_c, neg_hbm.at[pl.ds(base, CHUNK)])

        issue(0, 0)
        issue(1, 1)
        cu.wait()

        def pair_body(p, _):
            for buf in range(2):
                c = 2 * p + buf
                drain(buf)
                compute(c, buf)

                @pl.when(p < n_chunks // 2 - 1)
                def _():
                    issue(c + 2, buf)

            return 0

        lax.fori_loop(0, n_chunks // 2, pair_body, 0)

    return sc_call


def kernel(user_inputs, pos_inputs, neg_inputs, user_table, item_table):
    ui = jnp.squeeze(user_inputs, axis=-1)
    pi = jnp.squeeze(pos_inputs, axis=-1)
    ni = jnp.squeeze(neg_inputs, axis=-1)
    ut3 = user_table.reshape(user_table.shape[0], 1, D)
    u_flat = _sc_user_gather_call()(ui, ut3)
    it2 = item_table.reshape(item_table.shape[0] // 2, BLK)
    pos, neg = _sc_score_call()(
        pi >> 1, ni >> 1, (pi & 1) * D, (ni & 1) * D, it2, u_flat)
    return (pos[:, None], neg[:, None])
